# baseline (speedup 1.0000x reference)
import functools

import jax
import jax.numpy as jnp
from jax import lax
from jax.experimental import pallas as pl
from jax.experimental.pallas import tpu as pltpu
from jax.experimental.pallas import tpu_sc as plsc

D = 768
B = 4
S = 4096
NC = 2
NS = 16
NW = NC * NS
S_PER_W = S // NW
C = 32
NCHUNK = S_PER_W // C
LANES = 16
VECS_PER_ROW = D // LANES


def _emb_kernel(x_hbm, tok_hbm, pos_hbm, out_hbm,
                pos_v, tok_v, idx_v,
                g0, g1, g2, g3, s0_, s1_, s2_, s3_):
    gsems = (g0, g1, g2, g3)
    ssems = (s0_, s1_, s2_, s3_)
    wid = lax.axis_index("s") * NC + lax.axis_index("c")
    s_base = wid * S_PER_W

    def chunk_body(c, carry):
        s0 = s_base + c * C
        @pl.when(c > 0)
        def _():
            for b in range(B):
                pltpu.make_async_copy(
                    tok_v.at[b], out_hbm.at[b, pl.ds(s0, C)], ssems[b]
                ).wait()

        gat_cps = []
        for b in range(B):
            pltpu.sync_copy(x_hbm.at[b, pl.ds(s0, C)], idx_v.at[b])
            cp = pltpu.async_copy(tok_hbm.at[idx_v.at[b]], tok_v.at[b],
                                  gsems[b])
            gat_cps.append(cp)
        pltpu.sync_copy(pos_hbm.at[pl.ds(s0, C)], pos_v)
        for b in range(B):
            gat_cps[b].wait()

        def add_rows(r, _):
            for j in range(VECS_PER_ROW):
                sl = pl.ds(j * LANES, LANES)
                pv = pos_v[r, sl]
                for b in range(B):
                    tok_v[b, r, sl] = tok_v[b, r, sl] + pv
            return 0

        lax.fori_loop(0, C, add_rows, 0)

        for b in range(B):
            pltpu.async_copy(tok_v.at[b], out_hbm.at[b, pl.ds(s0, C)],
                             ssems[b])
        return carry

    lax.fori_loop(0, NCHUNK, chunk_body, 0)
    s_last = s_base + (NCHUNK - 1) * C
    for b in range(B):
        pltpu.make_async_copy(
            tok_v.at[b], out_hbm.at[b, pl.ds(s_last, C)], ssems[b]
        ).wait()


@jax.jit
def _emb(x, token_table, pos_table):
    mesh = plsc.VectorSubcoreMesh(core_axis_name="c", subcore_axis_name="s")
    kern = functools.partial(
        pl.kernel,
        mesh=mesh,
        out_type=jax.ShapeDtypeStruct((B, S, D), jnp.float32),
        scratch_types=[
            pltpu.VMEM((C, D), jnp.float32),
            pltpu.VMEM((B, C, D), jnp.float32),
            pltpu.VMEM((B, C), jnp.int32),
            pltpu.SemaphoreType.DMA,
            pltpu.SemaphoreType.DMA,
            pltpu.SemaphoreType.DMA,
            pltpu.SemaphoreType.DMA,
            pltpu.SemaphoreType.DMA,
            pltpu.SemaphoreType.DMA,
            pltpu.SemaphoreType.DMA,
            pltpu.SemaphoreType.DMA,
        ],
    )(_emb_kernel)
    return kern(x, token_table, pos_table)


def kernel(x, token_table, pos_table):
    return _emb(x.astype(jnp.int32), token_table, pos_table)


# speedup vs baseline: 2.0520x; 2.0520x over previous
import functools

import jax
import jax.numpy as jnp
from jax import lax
from jax.experimental import pallas as pl
from jax.experimental.pallas import tpu as pltpu
from jax.experimental.pallas import tpu_sc as plsc

D = 768
B = 4
S = 4096
NC = 2
NS = 16
NW = NC * NS
S_PER_W = S // NW
C = 8
NCHUNK = S_PER_W // C
NBUF = 4
LANES = 16
VECS_PER_ROW = D // LANES


def _emb_kernel(x_hbm, tok_hbm, pos_hbm, out_hbm,
                pos_v, tok_v, idx_v,
                p0, p1, p2, p3, g0, g1, g2, g3, s0_, s1_, s2_, s3_):
    psems = (p0, p1, p2, p3)
    gsems = (g0, g1, g2, g3)
    ssems = (s0_, s1_, s2_, s3_)
    wid = lax.axis_index("s") * NC + lax.axis_index("c")
    s_base = wid * S_PER_W

    for b in range(B):
        pltpu.sync_copy(x_hbm.at[b, pl.ds(s_base, S_PER_W)], idx_v.at[b])

    def fire(cc, q):
        s0 = s_base + cc * C
        pltpu.async_copy(pos_hbm.at[pl.ds(s0, C)], pos_v.at[q], psems[q])
        for b in range(B):
            pltpu.async_copy(tok_hbm.at[idx_v.at[b, pl.ds(cc * C, C)]],
                             tok_v.at[q, b], gsems[q])

    def wait_fire(q):
        pltpu.make_async_copy(pos_hbm.at[pl.ds(0, C)], pos_v.at[q],
                              psems[q]).wait()
        for b in range(B):
            pltpu.make_async_copy(tok_hbm.at[idx_v.at[b, pl.ds(0, C)]],
                                  tok_v.at[q, b], gsems[q]).wait()

    def fire_store(cc, q):
        s0 = s_base + cc * C
        for b in range(B):
            pltpu.async_copy(tok_v.at[q, b], out_hbm.at[b, pl.ds(s0, C)],
                             ssems[q])

    def wait_store(q):
        for b in range(B):
            pltpu.make_async_copy(tok_v.at[q, b],
                                  out_hbm.at[b, pl.ds(0, C)],
                                  ssems[q]).wait()

    def adds(q):
        def add_row(r, _):
            for j in range(VECS_PER_ROW):
                sl = pl.ds(j * LANES, LANES)
                pv = pos_v[q, r, sl]
                for b in range(B):
                    tok_v[q, b, r, sl] = tok_v[q, b, r, sl] + pv
            return 0

        lax.fori_loop(0, C, add_row, 0)

    fire(0, 0)
    fire(1, 1)

    def body(i, carry):
        for k in range(NBUF):
            c = NBUF * i + k
            q = (k + 2) % NBUF
            wait_fire(k)
            adds(k)
            cc2 = c + 2

            @pl.when(cc2 < NCHUNK)
            def _():
                @pl.when(cc2 >= NBUF)
                def _():
                    wait_store(q)

                fire(cc2, q)

            fire_store(c, k)
        return carry

    lax.fori_loop(0, NCHUNK // NBUF, body, 0)
    for q in range(NBUF):
        wait_store(q)


@jax.jit
def _emb(x, token_table, pos_table):
    mesh = plsc.VectorSubcoreMesh(core_axis_name="c", subcore_axis_name="s")
    kern = functools.partial(
        pl.kernel,
        mesh=mesh,
        out_type=jax.ShapeDtypeStruct((B, S, D), jnp.float32),
        scratch_types=[
            pltpu.VMEM((NBUF, C, D), jnp.float32),
            pltpu.VMEM((NBUF, B, C, D), jnp.float32),
            pltpu.VMEM((B, S_PER_W), jnp.int32),
        ] + [pltpu.SemaphoreType.DMA] * 12,
    )(_emb_kernel)
    return kern(x, token_table, pos_table)


def kernel(x, token_table, pos_table):
    return _emb(x.astype(jnp.int32), token_table, pos_table)


# speedup vs baseline: 2.0989x; 1.0229x over previous
import functools

import jax
import jax.numpy as jnp
from jax import lax
from jax.experimental import pallas as pl
from jax.experimental.pallas import tpu as pltpu
from jax.experimental.pallas import tpu_sc as plsc

D = 768
B = 4
S = 4096
NC = 2
NS = 16
NW = NC * NS
S_PER_W = S // NW
C = 8
NCHUNK = S_PER_W // C
NBUF = 4
LANES = 16
VECS_PER_ROW = D // LANES


def _emb_kernel(x_hbm, tok_hbm, pos_hbm, out_hbm,
                pos_v, tok_v, idx_v,
                p0, p1, p2, p3, g0, g1, g2, g3, s0_, s1_, s2_, s3_):
    psems = (p0, p1, p2, p3)
    gsems = (g0, g1, g2, g3)
    ssems = (s0_, s1_, s2_, s3_)
    wid = lax.axis_index("s") * NC + lax.axis_index("c")
    s_base = wid * S_PER_W

    idx_cps = [
        pltpu.async_copy(x_hbm.at[b, pl.ds(s_base, S_PER_W)], idx_v.at[b],
                         gsems[b])
        for b in range(B)
    ]
    for cp in idx_cps:
        cp.wait()

    def fire(cc, q):
        s0 = s_base + cc * C
        pltpu.async_copy(pos_hbm.at[pl.ds(s0, C)], pos_v.at[q], psems[q])
        for b in range(B):
            pltpu.async_copy(tok_hbm.at[idx_v.at[b, pl.ds(cc * C, C)]],
                             tok_v.at[q, b], gsems[q])

    def wait_fire(q):
        pltpu.make_async_copy(pos_hbm.at[pl.ds(0, C)], pos_v.at[q],
                              psems[q]).wait()
        for b in range(B):
            pltpu.make_async_copy(tok_hbm.at[idx_v.at[b, pl.ds(0, C)]],
                                  tok_v.at[q, b], gsems[q]).wait()

    def fire_store(cc, q):
        s0 = s_base + cc * C
        for b in range(B):
            pltpu.async_copy(tok_v.at[q, b], out_hbm.at[b, pl.ds(s0, C)],
                             ssems[q])

    def wait_store(q):
        for b in range(B):
            pltpu.make_async_copy(tok_v.at[q, b],
                                  out_hbm.at[b, pl.ds(0, C)],
                                  ssems[q]).wait()

    def adds(q):
        def add_row(r, _):
            for j in range(VECS_PER_ROW):
                sl = pl.ds(j * LANES, LANES)
                pv = pos_v[q, r, sl]
                for b in range(B):
                    tok_v[q, b, r, sl] = tok_v[q, b, r, sl] + pv
            return 0

        lax.fori_loop(0, C, add_row, 0)

    fire(0, 0)
    fire(1, 1)

    def body(i, carry):
        for k in range(NBUF):
            c = NBUF * i + k
            q = (k + 2) % NBUF
            wait_fire(k)
            adds(k)
            cc2 = c + 2

            @pl.when(cc2 < NCHUNK)
            def _():
                @pl.when(cc2 >= NBUF)
                def _():
                    wait_store(q)

                fire(cc2, q)

            fire_store(c, k)
        return carry

    lax.fori_loop(0, NCHUNK // NBUF, body, 0)
    for q in range(NBUF):
        wait_store(q)


@jax.jit
def _emb(x, token_table, pos_table):
    mesh = plsc.VectorSubcoreMesh(core_axis_name="c", subcore_axis_name="s")
    kern = functools.partial(
        pl.kernel,
        mesh=mesh,
        out_type=jax.ShapeDtypeStruct((B, S, D), jnp.float32),
        scratch_types=[
            pltpu.VMEM((NBUF, C, D), jnp.float32),
            pltpu.VMEM((NBUF, B, C, D), jnp.float32),
            pltpu.VMEM((B, S_PER_W), jnp.int32),
        ] + [pltpu.SemaphoreType.DMA] * 12,
    )(_emb_kernel)
    return kern(x, token_table, pos_table)


def kernel(x, token_table, pos_table):
    return _emb(x.astype(jnp.int32), token_table, pos_table)


# speedup vs baseline: 2.2117x; 1.0537x over previous
import functools

import jax
import jax.numpy as jnp
from jax import lax
from jax.experimental import pallas as pl
from jax.experimental.pallas import tpu as pltpu
from jax.experimental.pallas import tpu_sc as plsc

D = 768
B = 4
S = 4096
NC = 2
NS = 16
NW = NC * NS
S_PER_W = S // NW
C = 8
NCHUNK = S_PER_W // C
NBUF = 4
LANES = 16
VECS_PER_ROW = D // LANES


def _emb_kernel(x_hbm, tok_hbm, pos_hbm, out_hbm,
                pos_v, tok_v, idx_v,
                p0, p1, p2, p3, g0, g1, g2, g3, s0_, s1_, s2_, s3_):
    psems = (p0, p1, p2, p3)
    gsems = (g0, g1, g2, g3)
    ssems = (s0_, s1_, s2_, s3_)
    wid = lax.axis_index("s") * NC + lax.axis_index("c")
    s_base = wid * S_PER_W

    idx_cps = [
        pltpu.async_copy(x_hbm.at[b, pl.ds(s_base, S_PER_W)], idx_v.at[b],
                         gsems[b])
        for b in range(B)
    ]
    for cp in idx_cps:
        cp.wait()

    def fire(cc, q):
        s0 = s_base + cc * C
        pltpu.async_copy(pos_hbm.at[pl.ds(s0, C)], pos_v.at[q], psems[q])
        for b in range(B):
            pltpu.async_copy(tok_hbm.at[idx_v.at[b, pl.ds(cc * C, C)]],
                             tok_v.at[q, b], gsems[q])

    def wait_fire(q):
        pltpu.make_async_copy(pos_hbm.at[pl.ds(0, C)], pos_v.at[q],
                              psems[q]).wait()
        for b in range(B):
            pltpu.make_async_copy(tok_hbm.at[idx_v.at[b, pl.ds(0, C)]],
                                  tok_v.at[q, b], gsems[q]).wait()

    def fire_store(cc, q):
        s0 = s_base + cc * C
        for b in range(B):
            pltpu.async_copy(tok_v.at[q, b], out_hbm.at[b, pl.ds(s0, C)],
                             ssems[q])

    def wait_store(q):
        for b in range(B):
            pltpu.make_async_copy(tok_v.at[q, b],
                                  out_hbm.at[b, pl.ds(0, C)],
                                  ssems[q]).wait()

    def adds(q):
        def add_row(r, _):
            for j in range(VECS_PER_ROW):
                sl = pl.ds(j * LANES, LANES)
                pv = pos_v[q, r, sl]
                for b in range(B):
                    tok_v[q, b, r, sl] = tok_v[q, b, r, sl] + pv
            return 0

        lax.fori_loop(0, C, add_row, 0)

    fire(0, 0)
    fire(1, 1)

    def body(i, carry):
        for k in range(NBUF):
            c = NBUF * i + k
            q = (k + 2) % NBUF
            wait_fire(k)
            cc2 = c + 2

            @pl.when(cc2 < NCHUNK)
            def _():
                @pl.when(cc2 >= NBUF)
                def _():
                    wait_store(q)

                fire(cc2, q)

            adds(k)
            fire_store(c, k)
        return carry

    lax.fori_loop(0, NCHUNK // NBUF, body, 0)
    for q in range(NBUF):
        wait_store(q)


@jax.jit
def _emb(x, token_table, pos_table):
    mesh = plsc.VectorSubcoreMesh(core_axis_name="c", subcore_axis_name="s")
    kern = functools.partial(
        pl.kernel,
        mesh=mesh,
        out_type=jax.ShapeDtypeStruct((B, S, D), jnp.float32),
        scratch_types=[
            pltpu.VMEM((NBUF, C, D), jnp.float32),
            pltpu.VMEM((NBUF, B, C, D), jnp.float32),
            pltpu.VMEM((B, S_PER_W), jnp.int32),
        ] + [pltpu.SemaphoreType.DMA] * 12,
    )(_emb_kernel)
    return kern(x, token_table, pos_table)


def kernel(x, token_table, pos_table):
    return _emb(x.astype(jnp.int32), token_table, pos_table)
